# baseline (speedup 1.0000x reference)
import jax
import jax.numpy as jnp
from jax.experimental import pallas as pl
from jax.experimental.pallas import tpu as pltpu

_CH = 128
_NIN = 22
_NOUT = 6


def _body(sp_ref, rm_hbm, data_hbm, out_hbm,
          inbuf, outbuf, mask_i32, mask_f32,
          mask_sem, in_sem, out_sem):
    batch = data_hbm.shape[0]
    nsteps = batch // _CH
    sp = sp_ref[0]

    mask_cp = pltpu.make_async_copy(
        rm_hbm.at[pl.ds(sp, 1), :], mask_i32, mask_sem)
    mask_cp.start()

    def load(i):
        return pltpu.make_async_copy(
            data_hbm.at[pl.ds(i * _CH, _CH), :],
            inbuf.at[i % _NIN], in_sem.at[i % _NIN])

    def store(i):
        return pltpu.make_async_copy(
            outbuf.at[i % _NOUT],
            out_hbm.at[pl.ds(i * _CH, _CH), :], out_sem.at[i % _NOUT])

    for i in range(min(_NIN, nsteps)):
        load(i).start()

    mask_cp.wait()
    mask_f32[...] = mask_i32[...].astype(jnp.float32)

    for i in range(nsteps):
        load(i).wait()
        if i >= _NOUT:
            store(i - _NOUT).wait()
        outbuf[i % _NOUT] = inbuf[i % _NIN] * mask_f32[...]
        store(i).start()
        if i + _NIN < nsteps:
            load(i + _NIN).start()

    for i in range(max(0, nsteps - _NOUT), nsteps):
        store(i).wait()


def kernel(data, selected_param, region_map):
    batch, size = data.shape
    sp = jnp.asarray(selected_param, jnp.int32).reshape((1,))
    return pl.pallas_call(
        _body,
        in_specs=[
            pl.BlockSpec(memory_space=pltpu.MemorySpace.SMEM),
            pl.BlockSpec(memory_space=pl.ANY),
            pl.BlockSpec(memory_space=pl.ANY),
        ],
        out_specs=pl.BlockSpec(memory_space=pl.ANY),
        out_shape=jax.ShapeDtypeStruct((batch, size), jnp.float32),
        scratch_shapes=[
            pltpu.VMEM((_NIN, _CH, size), jnp.float32),
            pltpu.VMEM((_NOUT, _CH, size), jnp.float32),
            pltpu.VMEM((1, size), jnp.int32),
            pltpu.VMEM((1, size), jnp.float32),
            pltpu.SemaphoreType.DMA,
            pltpu.SemaphoreType.DMA((_NIN,)),
            pltpu.SemaphoreType.DMA((_NOUT,)),
        ],
    )(sp, region_map, data)


# speedup vs baseline: 1.0041x; 1.0041x over previous
import jax
import jax.numpy as jnp
from jax.experimental import pallas as pl
from jax.experimental.pallas import tpu as pltpu

_CH = 512
_NIN = 5
_NOUT = 2


def _body(sp_ref, rm_hbm, data_hbm, out_hbm,
          inbuf, outbuf, mask_i32, mask_f32,
          mask_sem, in_sem, out_sem):
    batch = data_hbm.shape[0]
    nsteps = batch // _CH
    sp = sp_ref[0]

    mask_cp = pltpu.make_async_copy(
        rm_hbm.at[pl.ds(sp, 1), :], mask_i32, mask_sem)
    mask_cp.start()

    def load(i):
        return pltpu.make_async_copy(
            data_hbm.at[pl.ds(i * _CH, _CH), :],
            inbuf.at[i % _NIN], in_sem.at[i % _NIN])

    def store(i):
        return pltpu.make_async_copy(
            outbuf.at[i % _NOUT],
            out_hbm.at[pl.ds(i * _CH, _CH), :], out_sem.at[i % _NOUT])

    for i in range(min(_NIN, nsteps)):
        load(i).start()

    mask_cp.wait()
    mask_f32[...] = mask_i32[...].astype(jnp.float32)

    for i in range(nsteps):
        load(i).wait()
        if i >= _NOUT:
            store(i - _NOUT).wait()
        outbuf[i % _NOUT] = inbuf[i % _NIN] * mask_f32[...]
        store(i).start()
        if i + _NIN < nsteps:
            load(i + _NIN).start()

    for i in range(max(0, nsteps - _NOUT), nsteps):
        store(i).wait()


def kernel(data, selected_param, region_map):
    batch, size = data.shape
    sp = jnp.asarray(selected_param, jnp.int32).reshape((1,))
    return pl.pallas_call(
        _body,
        in_specs=[
            pl.BlockSpec(memory_space=pltpu.MemorySpace.SMEM),
            pl.BlockSpec(memory_space=pl.ANY),
            pl.BlockSpec(memory_space=pl.ANY),
        ],
        out_specs=pl.BlockSpec(memory_space=pl.ANY),
        out_shape=jax.ShapeDtypeStruct((batch, size), jnp.float32),
        scratch_shapes=[
            pltpu.VMEM((_NIN, _CH, size), jnp.float32),
            pltpu.VMEM((_NOUT, _CH, size), jnp.float32),
            pltpu.VMEM((1, size), jnp.int32),
            pltpu.VMEM((1, size), jnp.float32),
            pltpu.SemaphoreType.DMA,
            pltpu.SemaphoreType.DMA((_NIN,)),
            pltpu.SemaphoreType.DMA((_NOUT,)),
        ],
    )(sp, region_map, data)
